# baseline (speedup 1.0000x reference)
import functools

import jax
import jax.numpy as jnp
from jax import lax
from jax.experimental import pallas as pl
from jax.experimental.pallas import tpu as pltpu
from jax.experimental.pallas import tpu_sc as plsc

N_FIELDS = 13
VOCAB = 100000
DIM = 16
BATCH = 16384

NC = 2
NS = 16
NW = NC * NS
SPW = BATCH // NW
C = 128
NCH = SPW // C
NCHG = BATCH // C


def _sc_kernel(qu, bu, qi, bi, ixu, ixi, out,
               idxv_u, idxv_i, ru, ri, bru, bri, outv, sem):
    wid = lax.axis_index("s") * NC + lax.axis_index("c")

    def chunk_body(c, carry):
        cg = wid * NCH + c
        base = wid * SPW + c * C
        pltpu.sync_copy(ixu.at[cg], idxv_u)
        pltpu.sync_copy(ixi.at[cg], idxv_i)
        descs = []
        for f in range(N_FIELDS):
            descs.append(pltpu.async_copy(qu.at[idxv_u.at[f]], ru.at[f], sem))
            descs.append(pltpu.async_copy(qi.at[idxv_i.at[f]], ri.at[f], sem))
            descs.append(pltpu.async_copy(bu.at[idxv_u.at[f]], bru.at[f], sem))
            descs.append(pltpu.async_copy(bi.at[idxv_i.at[f]], bri.at[f], sem))
        for d in descs:
            d.wait()

        lane = lax.iota(jnp.int32, 16)
        perms = [lane ^ (1 << k) for k in range(4)]

        def gbody(g, _):
            o = g * 16
            acc = jnp.zeros((16,), jnp.float32)
            for l in range(16):
                s = o + l
                au = ru[0, s, :]
                ai = ri[0, s, :]
                for f in range(1, N_FIELDS):
                    au = au + ru[f, s, :]
                    ai = ai + ri[f, s, :]
                t = au * ai
                for p in perms:
                    t = t + t.at[p].get(mode="promise_in_bounds")
                acc = jnp.where(lane == l, t, acc)
            for f in range(N_FIELDS):
                acc = acc + bru[f, pl.ds(o, 16)] + bri[f, pl.ds(o, 16)]
            outv[pl.ds(o, 16)] = acc
            return 0

        lax.fori_loop(0, C // 16, gbody, 0)
        pltpu.sync_copy(outv, out.at[pl.ds(base, C)])
        return carry

    lax.fori_loop(0, NCH, chunk_body, 0)


@functools.partial(jax.jit, static_argnames=())
def kernel(Q_user, B_user, Q_item, B_item, x_user, x_item):
    offs = (jnp.arange(N_FIELDS, dtype=jnp.int32) * VOCAB)[None, :]
    fu = x_user + offs
    fi = x_item + offs
    ixu = fu.reshape(NCHG, C, N_FIELDS).transpose(0, 2, 1)
    ixi = fi.reshape(NCHG, C, N_FIELDS).transpose(0, 2, 1)
    qu = Q_user.reshape(N_FIELDS * VOCAB, DIM)
    qi = Q_item.reshape(N_FIELDS * VOCAB, DIM)
    bu = B_user.reshape(N_FIELDS * VOCAB)
    bi = B_item.reshape(N_FIELDS * VOCAB)

    mesh = plsc.VectorSubcoreMesh(core_axis_name="c", subcore_axis_name="s")
    run = pl.kernel(
        _sc_kernel,
        out_type=jax.ShapeDtypeStruct((BATCH,), jnp.float32),
        mesh=mesh,
        compiler_params=pltpu.CompilerParams(use_tc_tiling_on_sc=False),
        scratch_types=[
            pltpu.VMEM((N_FIELDS, C), jnp.int32),
            pltpu.VMEM((N_FIELDS, C), jnp.int32),
            pltpu.VMEM((N_FIELDS, C, DIM), jnp.float32),
            pltpu.VMEM((N_FIELDS, C, DIM), jnp.float32),
            pltpu.VMEM((N_FIELDS, C), jnp.float32),
            pltpu.VMEM((N_FIELDS, C), jnp.float32),
            pltpu.VMEM((C,), jnp.float32),
            pltpu.SemaphoreType.DMA,
        ],
    )
    r = run(qu, bu, qi, bi, ixu, ixi)
    return r.reshape(BATCH, 1)


# speedup vs baseline: 1.0105x; 1.0105x over previous
import functools

import jax
import jax.numpy as jnp
from jax import lax
from jax.experimental import pallas as pl
from jax.experimental.pallas import tpu as pltpu
from jax.experimental.pallas import tpu_sc as plsc

N_FIELDS = 13
VOCAB = 100000
DIM = 16
BATCH = 16384

NC = 2
NS = 16
NW = NC * NS
SPW = BATCH // NW
C = 128
NCH = SPW // C
NG = C // 16


def _sc_kernel(qu, bu, qi, bi, ixu, ixi, dummy, out,
               ixu_v, ixi_v, ru, ri, bru, bri, outv, sem):
    wid = lax.axis_index("s") * NC + lax.axis_index("c")
    base = wid * SPW

    pltpu.sync_copy(ixu.at[:, pl.ds(base, SPW)], ixu_v)
    pltpu.sync_copy(ixi.at[:, pl.ds(base, SPW)], ixi_v)

    def chunk_body(c, carry):
        co = c * C

        def fire(f, carry2):
            iu = ixu_v.at[f, pl.ds(co, C)]
            ii = ixi_v.at[f, pl.ds(co, C)]
            pltpu.async_copy(qu.at[f].at[iu], ru.at[f], sem)
            pltpu.async_copy(qi.at[f].at[ii], ri.at[f], sem)
            pltpu.async_copy(bu.at[f].at[iu], bru.at[f], sem)
            pltpu.async_copy(bi.at[f].at[ii], bri.at[f], sem)
            return carry2

        lax.fori_loop(0, N_FIELDS, fire, 0)
        pltpu.make_async_copy(dummy, ru, sem).wait()
        pltpu.make_async_copy(dummy, ri, sem).wait()
        pltpu.make_async_copy(dummy.at[:, :, 0], bru, sem).wait()
        pltpu.make_async_copy(dummy.at[:, :, 0], bri, sem).wait()

        lane = lax.iota(jnp.int32, 16)
        perms = [lane ^ (1 << k) for k in range(4)]

        def gbody(g, carry3):
            o = g * 16
            acc = jnp.zeros((16,), jnp.float32)
            for l in range(16):
                s = o + l
                au = ru[0, s, :]
                ai = ri[0, s, :]
                for f in range(1, N_FIELDS):
                    au = au + ru[f, s, :]
                    ai = ai + ri[f, s, :]
                t = au * ai
                for p in perms:
                    t = t + t.at[p].get(mode="promise_in_bounds")
                acc = jnp.where(lane == l, t, acc)
            for f in range(N_FIELDS):
                acc = acc + bru[f, pl.ds(o, 16)] + bri[f, pl.ds(o, 16)]
            outv[pl.ds(co + o, 16)] = acc
            return carry3

        lax.fori_loop(0, NG, gbody, 0)
        return carry

    lax.fori_loop(0, NCH, chunk_body, 0)
    pltpu.sync_copy(outv, out.at[pl.ds(base, SPW)])


@functools.partial(jax.jit, static_argnames=())
def kernel(Q_user, B_user, Q_item, B_item, x_user, x_item):
    bu = B_user.reshape(N_FIELDS, VOCAB)
    bi = B_item.reshape(N_FIELDS, VOCAB)
    ixu = x_user.T
    ixi = x_item.T

    mesh = plsc.VectorSubcoreMesh(core_axis_name="c", subcore_axis_name="s")
    run = pl.kernel(
        _sc_kernel,
        out_type=jax.ShapeDtypeStruct((BATCH,), jnp.float32),
        mesh=mesh,
        compiler_params=pltpu.CompilerParams(use_tc_tiling_on_sc=False),
        scratch_types=[
            pltpu.VMEM((N_FIELDS, SPW), jnp.int32),
            pltpu.VMEM((N_FIELDS, SPW), jnp.int32),
            pltpu.VMEM((N_FIELDS, C, DIM), jnp.float32),
            pltpu.VMEM((N_FIELDS, C, DIM), jnp.float32),
            pltpu.VMEM((N_FIELDS, C), jnp.float32),
            pltpu.VMEM((N_FIELDS, C), jnp.float32),
            pltpu.VMEM((SPW,), jnp.float32),
            pltpu.SemaphoreType.DMA,
        ],
    )
    dummy = jnp.zeros((N_FIELDS, C, DIM), jnp.float32)
    r = run(Q_user, bu, Q_item, bi, ixu, ixi, dummy)
    return r.reshape(BATCH, 1)


# speedup vs baseline: 1.5992x; 1.5825x over previous
import functools

import jax
import jax.numpy as jnp
from jax import lax
from jax.experimental import pallas as pl
from jax.experimental.pallas import tpu as pltpu
from jax.experimental.pallas import tpu_sc as plsc

N_FIELDS = 13
VOCAB = 100000
DIM = 16
BATCH = 16384

NC = 2
NS = 16
NW = NC * NS
SPW = BATCH // NW
C = 128
NCH = SPW // C
NG = C // 16
ROWS = N_FIELDS * DIM


def _sc_kernel(qu, bu, qi, bi, ixu, ixi, dummy, out,
               ixu_v, ixi_v, ru, ri, bru, bri, outv, sem):
    wid = lax.axis_index("s") * NC + lax.axis_index("c")
    base = wid * SPW

    pltpu.sync_copy(ixu.at[:, pl.ds(base, SPW)], ixu_v)
    pltpu.sync_copy(ixi.at[:, pl.ds(base, SPW)], ixi_v)

    def chunk_body(c, carry):
        co = c * C

        def fire(f, carry2):
            iu = ixu_v.at[f, pl.ds(co, C)]
            ii = ixi_v.at[f, pl.ds(co, C)]
            for d in range(DIM):
                r = f * DIM + d
                pltpu.async_copy(qu.at[r].at[iu], ru.at[r], sem)
                pltpu.async_copy(qi.at[r].at[ii], ri.at[r], sem)
            pltpu.async_copy(bu.at[f].at[iu], bru.at[f], sem)
            pltpu.async_copy(bi.at[f].at[ii], bri.at[f], sem)
            return carry2

        lax.fori_loop(0, N_FIELDS, fire, 0)
        pltpu.make_async_copy(dummy, ru, sem).wait()
        pltpu.make_async_copy(dummy, ri, sem).wait()
        pltpu.make_async_copy(dummy.at[pl.ds(0, N_FIELDS)], bru, sem).wait()
        pltpu.make_async_copy(dummy.at[pl.ds(0, N_FIELDS)], bri, sem).wait()

        def gbody(g, carry3):
            o = g * 16
            acc = jnp.zeros((16,), jnp.float32)
            for d in range(DIM):
                quv = ru[d, pl.ds(o, 16)]
                qiv = ri[d, pl.ds(o, 16)]
                for f in range(1, N_FIELDS):
                    quv = quv + ru[f * DIM + d, pl.ds(o, 16)]
                    qiv = qiv + ri[f * DIM + d, pl.ds(o, 16)]
                acc = acc + quv * qiv
            for f in range(N_FIELDS):
                acc = acc + bru[f, pl.ds(o, 16)] + bri[f, pl.ds(o, 16)]
            outv[pl.ds(co + o, 16)] = acc
            return carry3

        lax.fori_loop(0, NG, gbody, 0)
        return carry

    lax.fori_loop(0, NCH, chunk_body, 0)
    pltpu.sync_copy(outv, out.at[pl.ds(base, SPW)])


@functools.partial(jax.jit, static_argnames=())
def kernel(Q_user, B_user, Q_item, B_item, x_user, x_item):
    qu = Q_user.transpose(0, 2, 1).reshape(ROWS, VOCAB)
    qi = Q_item.transpose(0, 2, 1).reshape(ROWS, VOCAB)
    bu = B_user.reshape(N_FIELDS, VOCAB)
    bi = B_item.reshape(N_FIELDS, VOCAB)
    ixu = x_user.T
    ixi = x_item.T

    mesh = plsc.VectorSubcoreMesh(core_axis_name="c", subcore_axis_name="s")
    run = pl.kernel(
        _sc_kernel,
        out_type=jax.ShapeDtypeStruct((BATCH,), jnp.float32),
        mesh=mesh,
        compiler_params=pltpu.CompilerParams(use_tc_tiling_on_sc=False),
        scratch_types=[
            pltpu.VMEM((N_FIELDS, SPW), jnp.int32),
            pltpu.VMEM((N_FIELDS, SPW), jnp.int32),
            pltpu.VMEM((ROWS, C), jnp.float32),
            pltpu.VMEM((ROWS, C), jnp.float32),
            pltpu.VMEM((N_FIELDS, C), jnp.float32),
            pltpu.VMEM((N_FIELDS, C), jnp.float32),
            pltpu.VMEM((SPW,), jnp.float32),
            pltpu.SemaphoreType.DMA,
        ],
    )
    dummy = jnp.zeros((ROWS, C), jnp.float32)
    r = run(qu, bu, qi, bi, ixu, ixi, dummy)
    return r.reshape(BATCH, 1)


# speedup vs baseline: 2.2722x; 1.4209x over previous
import functools

import jax
import jax.numpy as jnp
from jax import lax
from jax.experimental import pallas as pl
from jax.experimental.pallas import tpu as pltpu
from jax.experimental.pallas import tpu_sc as plsc

N_FIELDS = 13
VOCAB = 100000
VPAD = 100096
DIM = 16
BATCH = 16384

NC = 2
NS = 16
NW = NC * NS
SPW = BATCH // NW
C = 128
NCH = SPW // C
NG = C // 16
ROWS = N_FIELDS * DIM


def _user_kernel(qu, bu, ixu, dummy, qaccT, bua,
                 ixu_v, ru, bru, qa_v, ba_v, sem):
    wid = lax.axis_index("s") * NC + lax.axis_index("c")
    base = wid * SPW
    pltpu.sync_copy(ixu.at[:, pl.ds(base, SPW)], ixu_v)

    def chunk_body(c, carry):
        co = c * C

        def fire(f, carry2):
            iu = ixu_v.at[f, pl.ds(co, C)]
            for d in range(DIM):
                r = f * DIM + d
                pltpu.async_copy(qu.at[r].at[iu], ru.at[r], sem)
            pltpu.async_copy(bu.at[f].at[iu], bru.at[f], sem)
            return carry2

        lax.fori_loop(0, N_FIELDS, fire, 0)
        pltpu.make_async_copy(dummy, ru, sem).wait()
        pltpu.make_async_copy(dummy.at[pl.ds(0, N_FIELDS)], bru, sem).wait()

        def gbody(g, carry3):
            o = g * 16
            for d in range(DIM):
                quv = ru[d, pl.ds(o, 16)]
                for f in range(1, N_FIELDS):
                    quv = quv + ru[f * DIM + d, pl.ds(o, 16)]
                qa_v[d, pl.ds(co + o, 16)] = quv
            bacc = bru[0, pl.ds(o, 16)]
            for f in range(1, N_FIELDS):
                bacc = bacc + bru[f, pl.ds(o, 16)]
            ba_v[pl.ds(co + o, 16)] = bacc
            return carry3

        lax.fori_loop(0, NG, gbody, 0)
        return carry

    lax.fori_loop(0, NCH, chunk_body, 0)
    pltpu.sync_copy(qa_v, qaccT.at[:, pl.ds(base, SPW)])
    pltpu.sync_copy(ba_v, bua.at[pl.ds(base, SPW)])


def _item_kernel(qi, bi, ixi, qaccT, bua, dummy, out,
                 ixi_v, ri, bri, qa_v, ba_v, outv, sem):
    wid = lax.axis_index("s") * NC + lax.axis_index("c")
    base = wid * SPW
    pltpu.sync_copy(ixi.at[:, pl.ds(base, SPW)], ixi_v)
    pltpu.sync_copy(qaccT.at[:, pl.ds(base, SPW)], qa_v)
    pltpu.sync_copy(bua.at[pl.ds(base, SPW)], ba_v)

    def chunk_body(c, carry):
        co = c * C

        def fire(f, carry2):
            ii = ixi_v.at[f, pl.ds(co, C)]
            for d in range(DIM):
                r = f * DIM + d
                pltpu.async_copy(qi.at[r].at[ii], ri.at[r], sem)
            pltpu.async_copy(bi.at[f].at[ii], bri.at[f], sem)
            return carry2

        lax.fori_loop(0, N_FIELDS, fire, 0)
        pltpu.make_async_copy(dummy, ri, sem).wait()
        pltpu.make_async_copy(dummy.at[pl.ds(0, N_FIELDS)], bri, sem).wait()

        def gbody(g, carry3):
            o = g * 16
            acc = ba_v[pl.ds(co + o, 16)]
            for f in range(N_FIELDS):
                acc = acc + bri[f, pl.ds(o, 16)]
            for d in range(DIM):
                qiv = ri[d, pl.ds(o, 16)]
                for f in range(1, N_FIELDS):
                    qiv = qiv + ri[f * DIM + d, pl.ds(o, 16)]
                acc = acc + qa_v[d, pl.ds(co + o, 16)] * qiv
            outv[pl.ds(co + o, 16)] = acc
            return carry3

        lax.fori_loop(0, NG, gbody, 0)
        return carry

    lax.fori_loop(0, NCH, chunk_body, 0)
    pltpu.sync_copy(outv, out.at[pl.ds(base, SPW)])


@functools.partial(jax.jit, static_argnames=())
def kernel(Q_user, B_user, Q_item, B_item, x_user, x_item):
    qu = Q_user.transpose(0, 2, 1).reshape(ROWS, VOCAB)
    qi = Q_item.transpose(0, 2, 1).reshape(ROWS, VOCAB)
    bu = jnp.pad(B_user.reshape(N_FIELDS, VOCAB), ((0, 0), (0, VPAD - VOCAB)))
    bi = jnp.pad(B_item.reshape(N_FIELDS, VOCAB), ((0, 0), (0, VPAD - VOCAB)))
    ixu = x_user.T
    ixi = x_item.T
    dummy = jnp.zeros((ROWS, C), jnp.float32)

    mesh = plsc.VectorSubcoreMesh(core_axis_name="c", subcore_axis_name="s")
    cp = pltpu.CompilerParams(use_tc_tiling_on_sc=False)

    run_user = pl.kernel(
        _user_kernel,
        out_type=(
            jax.ShapeDtypeStruct((DIM, BATCH), jnp.float32),
            jax.ShapeDtypeStruct((BATCH,), jnp.float32),
        ),
        mesh=mesh,
        compiler_params=cp,
        scratch_types=[
            pltpu.VMEM((N_FIELDS, SPW), jnp.int32),
            pltpu.VMEM((ROWS, C), jnp.float32),
            pltpu.VMEM((N_FIELDS, C), jnp.float32),
            pltpu.VMEM((DIM, SPW), jnp.float32),
            pltpu.VMEM((SPW,), jnp.float32),
            pltpu.SemaphoreType.DMA,
        ],
    )
    qaccT, bua = run_user(qu, bu, ixu, dummy)

    run_item = pl.kernel(
        _item_kernel,
        out_type=jax.ShapeDtypeStruct((BATCH,), jnp.float32),
        mesh=mesh,
        compiler_params=cp,
        scratch_types=[
            pltpu.VMEM((N_FIELDS, SPW), jnp.int32),
            pltpu.VMEM((ROWS, C), jnp.float32),
            pltpu.VMEM((N_FIELDS, C), jnp.float32),
            pltpu.VMEM((DIM, SPW), jnp.float32),
            pltpu.VMEM((SPW,), jnp.float32),
            pltpu.VMEM((SPW,), jnp.float32),
            pltpu.SemaphoreType.DMA,
        ],
    )
    r = run_item(qi, bi, ixi, qaccT, bua, dummy)
    return r.reshape(BATCH, 1)
